# baseline (speedup 1.0000x reference)
import jax
import jax.numpy as jnp
from jax import lax
from jax.experimental import pallas as pl
from jax.experimental.pallas import tpu as pltpu
from jax.experimental.pallas import tpu_sc as plsc

E = 320000
D = 128
L = 16
NC, NS = 2, 16
NW = NC * NS
E_PER_W = E // NW
C = 80
N_CHUNKS = E_PER_W // C
G = C // L


def _sc_body(hd, hs, eidx, out, eidx_v, didx_v, sidx_v, a_v, b_v,
             out_v, sem_a, sem_b, sem_o):
    wid = lax.axis_index("s") * NC + lax.axis_index("c")
    base = wid * E_PER_W

    pltpu.sync_copy(eidx.at[pl.ds(2 * base, 2 * E_PER_W)], eidx_v)

    lanes = lax.iota(jnp.int32, L)

    def deinterleave(i, slot):
        for g in range(G):
            pos = 2 * (i * C + g * L) + 2 * lanes
            didx_v[pl.ds(slot * C + g * L, L)] = plsc.load_gather(
                eidx_v, [pos])
            sidx_v[pl.ds(slot * C + g * L, L)] = plsc.load_gather(
                eidx_v, [pos + 1])

    def issue_gathers(slot):
        pltpu.async_copy(hd.at[didx_v.at[pl.ds(slot * C, C)]], a_v.at[slot],
                         sem_a.at[slot])
        pltpu.async_copy(hs.at[sidx_v.at[pl.ds(slot * C, C)]], b_v.at[slot],
                         sem_b.at[slot])

    deinterleave(0, 0)
    issue_gathers(0)

    def chunk_body(i, _):
        slot = lax.rem(i, 2)
        off = i * C

        @pl.when(i + 1 < N_CHUNKS)
        def _():
            nslot = lax.rem(i + 1, 2)
            deinterleave(i + 1, nslot)
            issue_gathers(nslot)

        pltpu.make_async_copy(hd.at[didx_v.at[pl.ds(slot * C, C)]],
                              a_v.at[slot], sem_a.at[slot]).wait()
        pltpu.make_async_copy(hs.at[sidx_v.at[pl.ds(slot * C, C)]],
                              b_v.at[slot], sem_b.at[slot]).wait()

        @pl.when(i >= 2)
        def _():
            pltpu.make_async_copy(
                out_v.at[slot], out.at[pl.ds(base + off - 2 * C, C)],
                sem_o.at[slot]).wait()

        def group_body(g, _):
            e0 = g * L
            vec = jnp.zeros((L,), jnp.float32)
            for l in range(L):
                e = e0 + l
                a2 = plsc.bitcast(a_v[slot, e, pl.ds(0, L)], jnp.bfloat16)
                b2 = plsc.bitcast(b_v[slot, e, pl.ds(0, L)], jnp.bfloat16)
                acc2 = a2 * b2
                for j in range(1, D // (2 * L)):
                    a2 = plsc.bitcast(a_v[slot, e, pl.ds(j * L, L)],
                                      jnp.bfloat16)
                    b2 = plsc.bitcast(b_v[slot, e, pl.ds(j * L, L)],
                                      jnp.bfloat16)
                    acc2 = acc2 + a2 * b2
                p0, p1 = plsc.unpack(acc2,
                                     format=plsc.PackFormat.INTERLEAVED,
                                     preferred_element_type=jnp.float32)
                s = lax.reduce_sum(p0 + p1, axes=(0,))
                vec = jnp.where(lanes == l, s, vec)
            out_v[slot, pl.ds(e0, L)] = vec
            return ()

        lax.fori_loop(0, G, group_body, (), unroll=False)
        pltpu.async_copy(out_v.at[slot], out.at[pl.ds(base + off, C)],
                         sem_o.at[slot])
        return ()

    lax.fori_loop(0, N_CHUNKS, chunk_body, (), unroll=False)

    for k in (N_CHUNKS - 2, N_CHUNKS - 1):
        pltpu.make_async_copy(out_v.at[k % 2],
                              out.at[pl.ds(base + k * C, C)],
                              sem_o.at[k % 2]).wait()


@jax.jit
def _link_predict(h_drug, h_disease, eidx):
    mesh = plsc.VectorSubcoreMesh(core_axis_name="c", subcore_axis_name="s",
                                  num_cores=NC, num_subcores=NS)
    return pl.kernel(
        _sc_body,
        out_type=jax.ShapeDtypeStruct((E,), jnp.float32),
        mesh=mesh,
        compiler_params=pltpu.CompilerParams(needs_layout_passes=False,
                                             use_tc_tiling_on_sc=False),
        scratch_types=[
            pltpu.VMEM((2 * E_PER_W,), jnp.int32),
            pltpu.VMEM((2 * C,), jnp.int32),
            pltpu.VMEM((2 * C,), jnp.int32),
            pltpu.VMEM((2, C, D // 2), jnp.int32),
            pltpu.VMEM((2, C, D // 2), jnp.int32),
            pltpu.VMEM((2, C), jnp.float32),
            pltpu.SemaphoreType.DMA((2,)),
            pltpu.SemaphoreType.DMA((2,)),
            pltpu.SemaphoreType.DMA((2,)),
        ],
    )(h_drug, h_disease, eidx)


def _pack_table(h):
    u = lax.bitcast_convert_type(h, jnp.uint32)
    lo, hi = u[:, : D // 2], u[:, D // 2 :]
    half = jnp.uint32(0x7FFF)
    rlo = lo + half + ((lo >> 16) & jnp.uint32(1))
    rhi = hi + half + ((hi >> 16) & jnp.uint32(1))
    packed = (rhi & jnp.uint32(0xFFFF0000)) | (rlo >> 16)
    return lax.bitcast_convert_type(packed, jnp.int32)


def kernel(h_drug, h_disease, edges):
    eidx = edges.astype(jnp.int32).reshape(-1)
    return _link_predict(_pack_table(h_drug), _pack_table(h_disease), eidx)


# speedup vs baseline: 2.5241x; 2.5241x over previous
import jax
import jax.numpy as jnp
from jax import lax
from jax.experimental import pallas as pl
from jax.experimental.pallas import tpu as pltpu
from jax.experimental.pallas import tpu_sc as plsc

E = 320000
V = 10000
D = 128
L = 16
NC, NS = 2, 16
NW = NC * NS
E_PER_W = E // NW
V_PER_S = V // NS
C = 80
N_CHUNKS = E_PER_W // C
G = C // L


def _sc_body(hd, hs, didx, sidx, out, hd_sh, hs_sh, didx_v, sidx_v,
             a_v, b_v, out_v, sem_a, sem_b, sem_o):
    sid = lax.axis_index("s")
    wid = sid * NC + lax.axis_index("c")
    base = wid * E_PER_W

    row0 = sid * V_PER_S
    pltpu.sync_copy(hd.at[pl.ds(row0, V_PER_S)], hd_sh.at[pl.ds(row0, V_PER_S)])
    pltpu.sync_copy(hs.at[pl.ds(row0, V_PER_S)], hs_sh.at[pl.ds(row0, V_PER_S)])
    pltpu.sync_copy(didx.at[pl.ds(base, E_PER_W)], didx_v)
    pltpu.sync_copy(sidx.at[pl.ds(base, E_PER_W)], sidx_v)
    plsc.subcore_barrier()

    lanes = lax.iota(jnp.int32, L)

    def issue_gathers(i, slot):
        off = i * C
        pltpu.async_copy(hd_sh.at[didx_v.at[pl.ds(off, C)]], a_v.at[slot],
                         sem_a.at[slot])
        pltpu.async_copy(hs_sh.at[sidx_v.at[pl.ds(off, C)]], b_v.at[slot],
                         sem_b.at[slot])

    issue_gathers(0, 0)

    def chunk_body(i, _):
        slot = lax.rem(i, 2)
        off = i * C

        @pl.when(i + 1 < N_CHUNKS)
        def _():
            issue_gathers(i + 1, lax.rem(i + 1, 2))

        pltpu.make_async_copy(hd_sh.at[didx_v.at[pl.ds(off, C)]],
                              a_v.at[slot], sem_a.at[slot]).wait()
        pltpu.make_async_copy(hs_sh.at[sidx_v.at[pl.ds(off, C)]],
                              b_v.at[slot], sem_b.at[slot]).wait()

        @pl.when(i >= 2)
        def _():
            pltpu.make_async_copy(
                out_v.at[slot], out.at[pl.ds(base + off - 2 * C, C)],
                sem_o.at[slot]).wait()

        def group_body(g, _):
            e0 = g * L
            vec = jnp.zeros((L,), jnp.float32)
            for l in range(L):
                e = e0 + l
                a2 = plsc.bitcast(a_v[slot, e, pl.ds(0, L)], jnp.bfloat16)
                b2 = plsc.bitcast(b_v[slot, e, pl.ds(0, L)], jnp.bfloat16)
                acc2 = a2 * b2
                for j in range(1, D // (2 * L)):
                    a2 = plsc.bitcast(a_v[slot, e, pl.ds(j * L, L)],
                                      jnp.bfloat16)
                    b2 = plsc.bitcast(b_v[slot, e, pl.ds(j * L, L)],
                                      jnp.bfloat16)
                    acc2 = acc2 + a2 * b2
                p0, p1 = plsc.unpack(acc2,
                                     format=plsc.PackFormat.INTERLEAVED,
                                     preferred_element_type=jnp.float32)
                s = lax.reduce_sum(p0 + p1, axes=(0,))
                vec = jnp.where(lanes == l, s, vec)
            out_v[slot, pl.ds(e0, L)] = vec
            return ()

        lax.fori_loop(0, G, group_body, (), unroll=False)
        pltpu.async_copy(out_v.at[slot], out.at[pl.ds(base + off, C)],
                         sem_o.at[slot])
        return ()

    lax.fori_loop(0, N_CHUNKS, chunk_body, (), unroll=False)

    for k in (N_CHUNKS - 2, N_CHUNKS - 1):
        pltpu.make_async_copy(out_v.at[k % 2],
                              out.at[pl.ds(base + k * C, C)],
                              sem_o.at[k % 2]).wait()


@jax.jit
def _link_predict(h_drug, h_disease, d_idx, dis_idx):
    mesh = plsc.VectorSubcoreMesh(core_axis_name="c", subcore_axis_name="s",
                                  num_cores=NC, num_subcores=NS)
    return pl.kernel(
        _sc_body,
        out_type=jax.ShapeDtypeStruct((E,), jnp.float32),
        mesh=mesh,
        compiler_params=pltpu.CompilerParams(needs_layout_passes=False,
                                             use_tc_tiling_on_sc=False),
        scratch_types=[
            pltpu.VMEM_SHARED((V, D // 2), jnp.int32),
            pltpu.VMEM_SHARED((V, D // 2), jnp.int32),
            pltpu.VMEM((E_PER_W,), jnp.int32),
            pltpu.VMEM((E_PER_W,), jnp.int32),
            pltpu.VMEM((2, C, D // 2), jnp.int32),
            pltpu.VMEM((2, C, D // 2), jnp.int32),
            pltpu.VMEM((2, C), jnp.float32),
            pltpu.SemaphoreType.DMA((2,)),
            pltpu.SemaphoreType.DMA((2,)),
            pltpu.SemaphoreType.DMA((2,)),
        ],
    )(h_drug, h_disease, d_idx, dis_idx)


def _pack_table(h):
    u = lax.bitcast_convert_type(h, jnp.uint32)
    lo, hi = u[:, : D // 2], u[:, D // 2 :]
    half = jnp.uint32(0x7FFF)
    rlo = lo + half + ((lo >> 16) & jnp.uint32(1))
    rhi = hi + half + ((hi >> 16) & jnp.uint32(1))
    packed = (rhi & jnp.uint32(0xFFFF0000)) | (rlo >> 16)
    return lax.bitcast_convert_type(packed, jnp.int32)


def kernel(h_drug, h_disease, edges):
    e32 = edges.astype(jnp.int32)
    return _link_predict(_pack_table(h_drug), _pack_table(h_disease),
                         e32[:, 0], e32[:, 1])
